# Initial kernel scaffold; baseline (speedup 1.0000x reference)
import jax
import jax.numpy as jnp
from jax.experimental import pallas as pl


def kernel(x, P):
    raise NotImplementedError("write your pallas kernel here")


# speedup vs baseline: 1.4546x; 1.4546x over previous
import jax
import jax.numpy as jnp
from jax.experimental import pallas as pl

_BLOCK_W = 256


def _add_kernel(x_ref, p_ref, o_ref):
    o_ref[...] = x_ref[...] + p_ref[...]


def kernel(x, P):
    B, W, D = x.shape
    grid = (W // _BLOCK_W, B)
    return pl.pallas_call(
        _add_kernel,
        grid=grid,
        in_specs=[
            pl.BlockSpec((1, _BLOCK_W, D), lambda i, j: (j, i, 0)),
            pl.BlockSpec((1, _BLOCK_W, D), lambda i, j: (0, i, 0)),
        ],
        out_specs=pl.BlockSpec((1, _BLOCK_W, D), lambda i, j: (j, i, 0)),
        out_shape=jax.ShapeDtypeStruct((B, W, D), x.dtype),
    )(x, P[None])


# speedup vs baseline: 1.9317x; 1.3280x over previous
import jax
import jax.numpy as jnp
from jax.experimental import pallas as pl

_BLOCK_W = 512


def _add_kernel(x_ref, p_ref, o_ref):
    o_ref[...] = x_ref[...] + p_ref[...]


def kernel(x, P):
    B, W, D = x.shape
    grid = (W // _BLOCK_W, B)
    return pl.pallas_call(
        _add_kernel,
        grid=grid,
        in_specs=[
            pl.BlockSpec((1, _BLOCK_W, D), lambda i, j: (j, i, 0)),
            pl.BlockSpec((1, _BLOCK_W, D), lambda i, j: (0, i, 0)),
        ],
        out_specs=pl.BlockSpec((1, _BLOCK_W, D), lambda i, j: (j, i, 0)),
        out_shape=jax.ShapeDtypeStruct((B, W, D), x.dtype),
    )(x, P[None])


# speedup vs baseline: 2.1189x; 1.0969x over previous
import jax
import jax.numpy as jnp
from jax.experimental import pallas as pl

_BLOCK_W = 1024


def _add_kernel(x_ref, p_ref, o_ref):
    o_ref[...] = x_ref[...] + p_ref[...]


def kernel(x, P):
    B, W, D = x.shape
    grid = (W // _BLOCK_W, B)
    return pl.pallas_call(
        _add_kernel,
        grid=grid,
        in_specs=[
            pl.BlockSpec((1, _BLOCK_W, D), lambda i, j: (j, i, 0)),
            pl.BlockSpec((1, _BLOCK_W, D), lambda i, j: (0, i, 0)),
        ],
        out_specs=pl.BlockSpec((1, _BLOCK_W, D), lambda i, j: (j, i, 0)),
        out_shape=jax.ShapeDtypeStruct((B, W, D), x.dtype),
    )(x, P[None])


# speedup vs baseline: 2.2746x; 1.0735x over previous
import jax
import jax.numpy as jnp
from jax.experimental import pallas as pl

_BLOCK_W = 2048


def _add_kernel(x_ref, p_ref, o_ref):
    o_ref[...] = x_ref[...] + p_ref[...]


def kernel(x, P):
    B, W, D = x.shape
    grid = (W // _BLOCK_W, B)
    return pl.pallas_call(
        _add_kernel,
        grid=grid,
        in_specs=[
            pl.BlockSpec((1, _BLOCK_W, D), lambda i, j: (j, i, 0)),
            pl.BlockSpec((1, _BLOCK_W, D), lambda i, j: (0, i, 0)),
        ],
        out_specs=pl.BlockSpec((1, _BLOCK_W, D), lambda i, j: (j, i, 0)),
        out_shape=jax.ShapeDtypeStruct((B, W, D), x.dtype),
    )(x, P[None])


# speedup vs baseline: 2.2821x; 1.0033x over previous
import jax
import jax.numpy as jnp
from jax.experimental import pallas as pl
from jax.experimental.pallas import tpu as pltpu

_BLOCK_W = 2048


def _add_kernel(x_ref, p_ref, o_ref):
    o_ref[...] = x_ref[...] + p_ref[...]


def kernel(x, P):
    B, W, D = x.shape
    grid = (W // _BLOCK_W, B)
    return pl.pallas_call(
        _add_kernel,
        grid=grid,
        in_specs=[
            pl.BlockSpec((1, _BLOCK_W, D), lambda i, j: (j, i, 0)),
            pl.BlockSpec((1, _BLOCK_W, D), lambda i, j: (0, i, 0)),
        ],
        out_specs=pl.BlockSpec((1, _BLOCK_W, D), lambda i, j: (j, i, 0)),
        out_shape=jax.ShapeDtypeStruct((B, W, D), x.dtype),
        compiler_params=pltpu.CompilerParams(
            dimension_semantics=("parallel", "parallel"),
        ),
    )(x, P[None])
